# Initial kernel scaffold; baseline (speedup 1.0000x reference)
import jax
import jax.numpy as jnp
from jax.experimental import pallas as pl


def kernel(X, W):
    raise NotImplementedError("write your pallas kernel here")


# speedup vs baseline: 4.0895x; 4.0895x over previous
import functools

import jax
import jax.numpy as jnp
from jax import lax
from jax.experimental import pallas as pl
from jax.experimental.pallas import tpu as pltpu
from jax.experimental.pallas import tpu_sc as plsc

C_TABLE = 100000
D = 64
B_TOTAL = 4096 * 200
IW = 128
N_IDX_ROWS = B_TOTAL // IW
ROWS_PER_CHUNK = 4

_info = plsc.get_sparse_core_info()
NC, NS = _info.num_cores, _info.num_subcores
NW = NC * NS
ROWS_PER_W = N_IDX_ROWS // NW
CHUNKS = ROWS_PER_W // ROWS_PER_CHUNK

_mesh = plsc.VectorSubcoreMesh(core_axis_name="c", subcore_axis_name="s")


@functools.partial(
    pl.kernel,
    mesh=_mesh,
    compiler_params=pltpu.CompilerParams(use_tc_tiling_on_sc=False),
    out_type=jax.ShapeDtypeStruct((N_IDX_ROWS, IW, D), jnp.float32),
    scratch_types=[
        pltpu.VMEM((ROWS_PER_W, IW), jnp.int32),
        pltpu.VMEM((ROWS_PER_CHUNK, IW, D), jnp.float32),
        pltpu.SemaphoreType.DMA,
    ],
)
def _emb_lookup(w_hbm, x_hbm, out_hbm, idx_v, rows_v, sem):
    wid = lax.axis_index("s") * NC + lax.axis_index("c")
    row0 = wid * ROWS_PER_W
    pltpu.sync_copy(x_hbm.at[pl.ds(row0, ROWS_PER_W)], idx_v)

    def chunk(g, carry):
        r0 = g * ROWS_PER_CHUNK
        cps = [
            pltpu.async_copy(w_hbm.at[idx_v.at[r0 + j]], rows_v.at[j], sem)
            for j in range(ROWS_PER_CHUNK)
        ]
        for cp in cps:
            cp.wait()
        pltpu.sync_copy(rows_v, out_hbm.at[pl.ds(row0 + r0, ROWS_PER_CHUNK)])
        return carry

    lax.fori_loop(0, CHUNKS, chunk, 0)


def kernel(X, W):
    x2d = X.reshape(N_IDX_ROWS, IW).astype(jnp.int32)
    out = _emb_lookup(W, x2d)
    return out.reshape(X.shape[0], X.shape[1], D)


# speedup vs baseline: 4.2365x; 1.0359x over previous
import functools

import jax
import jax.numpy as jnp
from jax import lax
from jax.experimental import pallas as pl
from jax.experimental.pallas import tpu as pltpu
from jax.experimental.pallas import tpu_sc as plsc

C_TABLE = 100000
D = 64
B_TOTAL = 4096 * 200
IW = 128
N_IDX_ROWS = B_TOTAL // IW
ROWS_PER_CHUNK = 4

_info = plsc.get_sparse_core_info()
NC, NS = _info.num_cores, _info.num_subcores
NW = NC * NS
ROWS_PER_W = N_IDX_ROWS // NW
CHUNKS = ROWS_PER_W // ROWS_PER_CHUNK

_mesh = plsc.VectorSubcoreMesh(core_axis_name="c", subcore_axis_name="s")


@functools.partial(
    pl.kernel,
    mesh=_mesh,
    compiler_params=pltpu.CompilerParams(use_tc_tiling_on_sc=False),
    out_type=jax.ShapeDtypeStruct((N_IDX_ROWS, IW, D), jnp.float32),
    scratch_types=[
        pltpu.VMEM((ROWS_PER_W, IW), jnp.int32),
        pltpu.VMEM((2, ROWS_PER_CHUNK, IW, D), jnp.float32),
        pltpu.SemaphoreType.DMA((2,)),
        pltpu.SemaphoreType.DMA((2,)),
    ],
)
def _emb_lookup(w_hbm, x_hbm, out_hbm, idx_v, rows_v, gsem, wsem):
    wid = lax.axis_index("s") * NC + lax.axis_index("c")
    row0 = wid * ROWS_PER_W
    pltpu.sync_copy(x_hbm.at[pl.ds(row0, ROWS_PER_W)], idx_v)

    def start_gathers(g, b):
        r0 = g * ROWS_PER_CHUNK
        for j in range(ROWS_PER_CHUNK):
            pltpu.async_copy(
                w_hbm.at[idx_v.at[r0 + j]], rows_v.at[b].at[j], gsem.at[b])

    def wait_gathers(g, b):
        r0 = g * ROWS_PER_CHUNK
        for j in range(ROWS_PER_CHUNK):
            pltpu.make_async_copy(
                w_hbm.at[idx_v.at[r0 + j]], rows_v.at[b].at[j],
                gsem.at[b]).wait()

    def wb_copy(g, b):
        return pltpu.make_async_copy(
            rows_v.at[b],
            out_hbm.at[pl.ds(row0 + g * ROWS_PER_CHUNK, ROWS_PER_CHUNK)],
            wsem.at[b])

    start_gathers(0, 0)

    def chunk(g, carry):
        b = lax.rem(g, 2)
        nb = 1 - b
        wait_gathers(g, b)
        wb_copy(g, b).start()

        @pl.when(g + 1 < CHUNKS)
        def _():
            @pl.when(g >= 1)
            def _():
                wb_copy(g - 1, nb).wait()
            start_gathers(g + 1, nb)

        return carry

    lax.fori_loop(0, CHUNKS, chunk, 0)
    wb_copy(CHUNKS - 2, lax.rem(CHUNKS - 2, 2)).wait()
    wb_copy(CHUNKS - 1, lax.rem(CHUNKS - 1, 2)).wait()


def kernel(X, W):
    x2d = X.reshape(N_IDX_ROWS, IW).astype(jnp.int32)
    out = _emb_lookup(W, x2d)
    return out.reshape(X.shape[0], X.shape[1], D)
